# baseline (speedup 1.0000x reference)
import functools

import jax
import jax.numpy as jnp
from jax import lax
from jax.experimental import pallas as pl
from jax.experimental.pallas import tpu as pltpu
from jax.experimental.pallas import tpu_sc as plsc

_INFO = plsc.get_sparse_core_info()
_NC, _NS = _INFO.num_cores, _INFO.num_subcores
_NW = _NC * _NS
_CHUNK = 128


@functools.partial(jax.jit, static_argnums=(3, 4))
def _gather_sc(idx, pe_x, pe_y, b_per_w, nchunk):
    B = idx.shape[1] * idx.shape[2] * idx.shape[3]
    D = pe_x.shape[1]
    mesh = plsc.VectorSubcoreMesh(core_axis_name="c", subcore_axis_name="s")

    @functools.partial(
        pl.kernel,
        mesh=mesh,
        compiler_params=pltpu.CompilerParams(use_tc_tiling_on_sc=False),
        out_type=[
            jax.ShapeDtypeStruct((B, D), jnp.float32),
            jax.ShapeDtypeStruct((B, D), jnp.float32),
        ],
        scratch_types=[
            pltpu.VMEM((nchunk, _CHUNK), jnp.int32),
            pltpu.VMEM((nchunk, _CHUNK), jnp.int32),
            pltpu.VMEM((b_per_w, D), jnp.float32),
            pltpu.VMEM((b_per_w, D), jnp.float32),
            pltpu.SemaphoreType.DMA,
        ],
    )
    def k(idx_hbm, pe_x_hbm, pe_y_hbm, out_x_hbm, out_y_hbm,
          idxx_v, idxy_v, rowsx_v, rowsy_v, sem):
        wid = lax.axis_index("s") * _NC + lax.axis_index("c")
        base = wid * b_per_w
        pltpu.sync_copy(idx_hbm.at[0, wid], idxx_v)
        pltpu.sync_copy(idx_hbm.at[1, wid], idxy_v)
        copies = []
        for j in range(nchunk):
            copies.append(pltpu.async_copy(
                pe_x_hbm.at[idxx_v.at[j]],
                rowsx_v.at[pl.ds(j * _CHUNK, _CHUNK)], sem))
            copies.append(pltpu.async_copy(
                pe_y_hbm.at[idxy_v.at[j]],
                rowsy_v.at[pl.ds(j * _CHUNK, _CHUNK)], sem))
        for c in copies:
            c.wait()
        pltpu.sync_copy(rowsx_v, out_x_hbm.at[pl.ds(base, b_per_w)])
        pltpu.sync_copy(rowsy_v, out_y_hbm.at[pl.ds(base, b_per_w)])

    return k(idx, pe_x, pe_y)


def kernel(xy_tensor, pe_x, pe_y):
    B = xy_tensor.shape[-1]
    b_per_w = B // _NW
    nchunk = b_per_w // _CHUNK
    idx = xy_tensor.reshape(2, _NW, nchunk, _CHUNK)
    return tuple(_gather_sc(idx, pe_x, pe_y, b_per_w, nchunk))
